# Initial kernel scaffold; baseline (speedup 1.0000x reference)
import jax
import jax.numpy as jnp
from jax.experimental import pallas as pl


def kernel(seq1, seq2, seq3, seq4, adj1, adj2, W_enc2, b_enc2, a_enc2, W_dec, b_dec, a_dec, W1, W2, W3, Wlin, blin, a_act, Wb1, bb1, Wb2, bb2, Wb3, bb3, Wb4, bb4):
    raise NotImplementedError("write your pallas kernel here")


# speedup vs baseline: 2.1560x; 2.1560x over previous
import jax
import jax.numpy as jnp
from jax.experimental import pallas as pl
from jax.experimental.pallas import tpu as pltpu

B, S, NIN, NH = 16384, 4, 128, 64
BB = 512
NB = B // BB


def _prelu(x, a):
    return jnp.maximum(x, 0.0) + a * jnp.minimum(x, 0.0)


def _fused_kernel(seq1_ref, seq2_ref, seq3_ref, seq4_ref, adj1_ref, adj2_ref,
                  W_enc2_ref, b_enc2_ref, a_enc2_ref,
                  W_dec_ref, b_dec_ref, a_dec_ref,
                  W1_ref, W2_ref, W3_ref, Wlin_ref, blin_ref, a_act_ref,
                  Wb1_ref, bb1_ref, Wb2_ref, bb2_ref,
                  Wb3_ref, bb3_ref, Wb4_ref, bb4_ref,
                  f1_ref, f2_ref, sc_ref, ex_ref):
    a1 = adj1_ref[...]
    a2 = adj2_ref[...]

    def dot(x, w):
        return jnp.dot(x, w, preferred_element_type=jnp.float32)

    def combine(a, fts):
        outs = []
        for i in range(S):
            acc = a[:, 4 * i:4 * i + 1] * fts[0]
            for j in range(1, S):
                acc = acc + a[:, 4 * i + j:4 * i + j + 1] * fts[j]
            outs.append(acc)
        return outs

    def gcn(seq_ref, a, W, bvec, alpha):
        fts = [dot(seq_ref[:, j * NIN:(j + 1) * NIN], W) for j in range(S)]
        return [_prelu(g + bvec, alpha) for g in combine(a, fts)]

    def encoder1(seq_ref, a):
        f = [dot(seq_ref[:, j * NIN:(j + 1) * NIN], W1_ref[...]) for j in range(S)]
        g = combine(a, f)
        g = combine(a, [dot(x, W2_ref[...]) for x in g])
        g = combine(a, [dot(x, W3_ref[...]) for x in g])
        blin = blin_ref[...]
        a_act = a_act_ref[...]
        return [_prelu(dot(x, Wlin_ref[...]) + blin, a_act) for x in g]

    h1 = gcn(seq1_ref, a1, W_enc2_ref[...], b_enc2_ref[...], a_enc2_ref[...])
    h2 = gcn(seq2_ref, a2, W_enc2_ref[...], b_enc2_ref[...], a_enc2_ref[...])

    third = jnp.float32(1.0 / 3.0)
    c1 = (h1[0] + h1[1] + h1[2]) * third
    mv1 = h1[3]
    ano1 = h1[2]
    c2 = (h2[0] + h2[1] + h2[2]) * third
    mv2 = h2[3]

    h3 = encoder1(seq3_ref, a1)
    h4 = encoder1(seq4_ref, a2)

    W_dec = W_dec_ref[...]
    b_dec = b_dec_ref[...]
    a_dec = a_dec_ref[...]
    d1 = combine(a1, [dot(x, W_dec) for x in h3])
    d2 = combine(a2, [dot(x, W_dec) for x in h4])
    for i in range(S):
        f1_ref[:, i * NIN:(i + 1) * NIN] = _prelu(d1[i] + b_dec, a_dec)
        f2_ref[:, i * NIN:(i + 1) * NIN] = _prelu(d2[i] + b_dec, a_dec)

    u1 = dot(mv1, Wb1_ref[...])
    u2 = dot(mv2, Wb2_ref[...])
    u3 = dot(mv1, Wb3_ref[...])
    u4 = dot(mv1, Wb4_ref[...])

    def shift_down(x):
        return jnp.concatenate([x[BB - 1:BB], x[:BB - 1]], axis=0)

    bb1 = bb1_ref[...]
    bb2 = bb2_ref[...]
    bb3 = bb3_ref[...]
    bb4 = bb4_ref[...]
    c1s = shift_down(c1)
    c2s = shift_down(c2)
    anos = shift_down(ano1)

    def rsum(x, y, b):
        return jnp.sum(x * y, axis=1, keepdims=True) + b

    sc_ref[...] = jnp.concatenate(
        [rsum(u1, c1, bb1), rsum(u1, c1s, bb1),
         rsum(u2, c2, bb2), rsum(u2, c2s, bb2),
         rsum(u3, ano1, bb3), rsum(u3, anos, bb3),
         rsum(u4, ano1, bb4), rsum(u4, anos, bb4)], axis=1)

    ex_ref[0] = jnp.concatenate(
        [u1[0:1], u2[0:1], u3[0:1], u4[0:1],
         c1[BB - 1:BB], c2[BB - 1:BB], ano1[BB - 1:BB], ano1[BB - 2:BB - 1]],
        axis=0)


def kernel(seq1, seq2, seq3, seq4, adj1, adj2,
           W_enc2, b_enc2, a_enc2, W_dec, b_dec, a_dec,
           W1, W2, W3, Wlin, blin, a_act,
           Wb1, bb1, Wb2, bb2, Wb3, bb3, Wb4, bb4):
    seqs = [x.reshape(B, S * NIN) for x in (seq1, seq2, seq3, seq4)]
    adjs = [x.reshape(B, S * S) for x in (adj1, adj2)]

    row = lambda i: (i, 0)
    whole = lambda i: (0, 0)
    seq_spec = pl.BlockSpec((BB, S * NIN), row)
    adj_spec = pl.BlockSpec((BB, S * S), row)

    def wspec(arr):
        return pl.BlockSpec(arr.shape, whole)

    weights = [W_enc2, b_enc2.reshape(1, NH), a_enc2.reshape(1, 1),
               W_dec, b_dec.reshape(1, NIN), a_dec.reshape(1, 1),
               W1, W2, W3, Wlin, blin.reshape(1, NH), a_act.reshape(1, 1),
               Wb1.reshape(NH, NH), bb1.reshape(1, 1),
               Wb2.reshape(NH, NH), bb2.reshape(1, 1),
               Wb3.reshape(NH, NH), bb3.reshape(1, 1),
               Wb4.reshape(NH, NH), bb4.reshape(1, 1)]

    f1o, f2o, sc, ex = pl.pallas_call(
        _fused_kernel,
        grid=(NB,),
        in_specs=[seq_spec] * 4 + [adj_spec] * 2 + [wspec(w) for w in weights],
        out_specs=[
            pl.BlockSpec((BB, S * NIN), row),
            pl.BlockSpec((BB, S * NIN), row),
            pl.BlockSpec((BB, 8), row),
            pl.BlockSpec((1, 8, NH), lambda i: (i, 0, 0)),
        ],
        out_shape=[
            jax.ShapeDtypeStruct((B, S * NIN), jnp.float32),
            jax.ShapeDtypeStruct((B, S * NIN), jnp.float32),
            jax.ShapeDtypeStruct((B, 8), jnp.float32),
            jax.ShapeDtypeStruct((NB, 8, NH), jnp.float32),
        ],
        compiler_params=pltpu.CompilerParams(
            dimension_semantics=("parallel",),
        ),
    )(*seqs, *adjs, *weights)

    u_first = ex[:, 0:4, :]
    c1_prev = jnp.roll(ex[:, 4, :], 1, axis=0)
    c2_prev = jnp.roll(ex[:, 5, :], 1, axis=0)
    ano_prev = jnp.roll(ex[:, 6, :], 1, axis=0)
    ano_prev = ano_prev.at[0].set(ex[-1, 7, :])
    fix1 = jnp.sum(u_first[:, 0] * c1_prev, axis=1) + bb1[0]
    fix2 = jnp.sum(u_first[:, 1] * c2_prev, axis=1) + bb2[0]
    fix3 = jnp.sum(u_first[:, 2] * ano_prev, axis=1) + bb3[0]
    fix4 = jnp.sum(u_first[:, 3] * ano_prev, axis=1) + bb4[0]

    sc3 = sc.reshape(NB, BB, 8)
    sc3 = (sc3.at[:, 0, 1].set(fix1).at[:, 0, 3].set(fix2)
              .at[:, 0, 5].set(fix3).at[:, 0, 7].set(fix4))
    scf = sc3.reshape(B, 8)

    def ret(ka, kb):
        return jnp.concatenate([scf[:, ka], scf[:, kb]])[:, None]

    return (ret(0, 1), ret(2, 3), ret(4, 5), ret(6, 7),
            f1o.reshape(B, S, NIN), f2o.reshape(B, S, NIN))


# speedup vs baseline: 2.1567x; 1.0003x over previous
import jax
import jax.numpy as jnp
from jax.experimental import pallas as pl
from jax.experimental.pallas import tpu as pltpu

B, S, NIN, NH = 16384, 4, 128, 64
BB = 512
NB = B // BB


def _prelu(x, a):
    return jnp.maximum(x, 0.0) + a * jnp.minimum(x, 0.0)


def _fused_kernel(seq1_ref, seq2_ref, seq3_ref, seq4_ref, adj1_ref, adj2_ref,
                  W_enc2_ref, b_enc2_ref, a_enc2_ref,
                  W_dec_ref, b_dec_ref, a_dec_ref,
                  W1_ref, W2_ref, W3_ref, Wlin_ref, blin_ref, a_act_ref,
                  Wb1_ref, bb1_ref, Wb2_ref, bb2_ref,
                  Wb3_ref, bb3_ref, Wb4_ref, bb4_ref,
                  f1_ref, f2_ref, sc_ref, ex_ref):
    a1 = adj1_ref[...]
    a2 = adj2_ref[...]

    def dot(x, w):
        return jnp.dot(x, w, preferred_element_type=jnp.float32)

    def combine(a, fts):
        outs = []
        for i in range(S):
            acc = a[:, 4 * i:4 * i + 1] * fts[0]
            for j in range(1, S):
                acc = acc + a[:, 4 * i + j:4 * i + j + 1] * fts[j]
            outs.append(acc)
        return outs

    def gcn(seq_ref, a, W, bvec, alpha):
        fts = [dot(seq_ref[:, j * NIN:(j + 1) * NIN], W) for j in range(S)]
        return [_prelu(g + bvec, alpha) for g in combine(a, fts)]

    def encoder1(seq_ref, a):
        f = [dot(seq_ref[:, j * NIN:(j + 1) * NIN], W1_ref[...]) for j in range(S)]
        g = combine(a, f)
        g = combine(a, [dot(x, W2_ref[...]) for x in g])
        g = combine(a, [dot(x, W3_ref[...]) for x in g])
        blin = blin_ref[...]
        a_act = a_act_ref[...]
        return [_prelu(dot(x, Wlin_ref[...]) + blin, a_act) for x in g]

    h1 = gcn(seq1_ref, a1, W_enc2_ref[...], b_enc2_ref[...], a_enc2_ref[...])
    h2 = gcn(seq2_ref, a2, W_enc2_ref[...], b_enc2_ref[...], a_enc2_ref[...])

    third = jnp.float32(1.0 / 3.0)
    c1 = (h1[0] + h1[1] + h1[2]) * third
    mv1 = h1[3]
    ano1 = h1[2]
    c2 = (h2[0] + h2[1] + h2[2]) * third
    mv2 = h2[3]

    h3 = encoder1(seq3_ref, a1)
    h4 = encoder1(seq4_ref, a2)

    W_dec = W_dec_ref[...]
    b_dec = b_dec_ref[...]
    a_dec = a_dec_ref[...]
    d1 = combine(a1, [dot(x, W_dec) for x in h3])
    d2 = combine(a2, [dot(x, W_dec) for x in h4])
    for i in range(S):
        f1_ref[:, i * NIN:(i + 1) * NIN] = _prelu(d1[i] + b_dec, a_dec)
        f2_ref[:, i * NIN:(i + 1) * NIN] = _prelu(d2[i] + b_dec, a_dec)

    u1 = dot(mv1, Wb1_ref[...])
    u2 = dot(mv2, Wb2_ref[...])
    u3 = dot(mv1, Wb3_ref[...])
    u4 = dot(mv1, Wb4_ref[...])

    def shift_down(x):
        return jnp.concatenate([x[BB - 1:BB], x[:BB - 1]], axis=0)

    bb1 = bb1_ref[...]
    bb2 = bb2_ref[...]
    bb3 = bb3_ref[...]
    bb4 = bb4_ref[...]
    c1s = shift_down(c1)
    c2s = shift_down(c2)
    anos = shift_down(ano1)

    def rsum(x, y, b):
        return jnp.sum(x * y, axis=1, keepdims=True) + b

    sc_ref[...] = jnp.concatenate(
        [rsum(u1, c1, bb1), rsum(u1, c1s, bb1),
         rsum(u2, c2, bb2), rsum(u2, c2s, bb2),
         rsum(u3, ano1, bb3), rsum(u3, anos, bb3),
         rsum(u4, ano1, bb4), rsum(u4, anos, bb4)], axis=1)

    ex_ref[0] = jnp.concatenate(
        [u1[0:1], u2[0:1], u3[0:1], u4[0:1],
         c1[BB - 1:BB], c2[BB - 1:BB], ano1[BB - 1:BB], ano1[BB - 2:BB - 1]],
        axis=0)


def kernel(seq1, seq2, seq3, seq4, adj1, adj2,
           W_enc2, b_enc2, a_enc2, W_dec, b_dec, a_dec,
           W1, W2, W3, Wlin, blin, a_act,
           Wb1, bb1, Wb2, bb2, Wb3, bb3, Wb4, bb4):
    seqs = [x.reshape(B, S * NIN) for x in (seq1, seq2, seq3, seq4)]
    adjs = [x.reshape(B, S * S) for x in (adj1, adj2)]

    row = lambda i: (i, 0)
    whole = lambda i: (0, 0)
    seq_spec = pl.BlockSpec((BB, S * NIN), row)
    adj_spec = pl.BlockSpec((BB, S * S), row)

    def wspec(arr):
        return pl.BlockSpec(arr.shape, whole)

    weights = [W_enc2, b_enc2.reshape(1, NH), a_enc2.reshape(1, 1),
               W_dec, b_dec.reshape(1, NIN), a_dec.reshape(1, 1),
               W1, W2, W3, Wlin, blin.reshape(1, NH), a_act.reshape(1, 1),
               Wb1.reshape(NH, NH), bb1.reshape(1, 1),
               Wb2.reshape(NH, NH), bb2.reshape(1, 1),
               Wb3.reshape(NH, NH), bb3.reshape(1, 1),
               Wb4.reshape(NH, NH), bb4.reshape(1, 1)]

    f1o, f2o, sc, ex = pl.pallas_call(
        _fused_kernel,
        grid=(NB,),
        in_specs=[seq_spec] * 4 + [adj_spec] * 2 + [wspec(w) for w in weights],
        out_specs=[
            pl.BlockSpec((BB, S * NIN), row),
            pl.BlockSpec((BB, S * NIN), row),
            pl.BlockSpec((BB, 8), row),
            pl.BlockSpec((1, 8, NH), lambda i: (i, 0, 0)),
        ],
        out_shape=[
            jax.ShapeDtypeStruct((B, S * NIN), jnp.float32),
            jax.ShapeDtypeStruct((B, S * NIN), jnp.float32),
            jax.ShapeDtypeStruct((B, 8), jnp.float32),
            jax.ShapeDtypeStruct((NB, 8, NH), jnp.float32),
        ],
        compiler_params=pltpu.CompilerParams(
            dimension_semantics=("arbitrary",),
        ),
    )(*seqs, *adjs, *weights)

    u_first = ex[:, 0:4, :]
    c1_prev = jnp.roll(ex[:, 4, :], 1, axis=0)
    c2_prev = jnp.roll(ex[:, 5, :], 1, axis=0)
    ano_prev = jnp.roll(ex[:, 6, :], 1, axis=0)
    ano_prev = ano_prev.at[0].set(ex[-1, 7, :])
    fix1 = jnp.sum(u_first[:, 0] * c1_prev, axis=1) + bb1[0]
    fix2 = jnp.sum(u_first[:, 1] * c2_prev, axis=1) + bb2[0]
    fix3 = jnp.sum(u_first[:, 2] * ano_prev, axis=1) + bb3[0]
    fix4 = jnp.sum(u_first[:, 3] * ano_prev, axis=1) + bb4[0]

    sc3 = sc.reshape(NB, BB, 8)
    sc3 = (sc3.at[:, 0, 1].set(fix1).at[:, 0, 3].set(fix2)
              .at[:, 0, 5].set(fix3).at[:, 0, 7].set(fix4))
    scf = sc3.reshape(B, 8)

    def ret(ka, kb):
        return jnp.concatenate([scf[:, ka], scf[:, kb]])[:, None]

    return (ret(0, 1), ret(2, 3), ret(4, 5), ret(6, 7),
            f1o.reshape(B, S, NIN), f2o.reshape(B, S, NIN))


# speedup vs baseline: 2.2186x; 1.0287x over previous
import jax
import jax.numpy as jnp
from jax.experimental import pallas as pl
from jax.experimental.pallas import tpu as pltpu

B, S, NIN, NH = 16384, 4, 128, 64
BB = 512
NB = B // BB


def _prelu(x, a):
    return jnp.maximum(x, 0.0) + a * jnp.minimum(x, 0.0)


def _fused_kernel(seq1_ref, seq2_ref, seq3_ref, seq4_ref, adj1_ref, adj2_ref,
                  W_enc2_ref, b_enc2_ref, a_enc2_ref,
                  W_dec_ref, b_dec_ref, a_dec_ref,
                  W1_ref, W2_ref, W3_ref, Wlin_ref, blin_ref, a_act_ref,
                  Wb1_ref, bb1_ref, Wb2_ref, bb2_ref,
                  Wb3_ref, bb3_ref, Wb4_ref, bb4_ref,
                  f1_ref, f2_ref, sc_ref, ex_ref):
    a1 = adj1_ref[...]
    a2 = adj2_ref[...]

    def dot(x, w):
        return jnp.dot(x, w, preferred_element_type=jnp.float32)

    def combine(a, fts):
        outs = []
        for i in range(S):
            acc = a[:, 4 * i:4 * i + 1] * fts[0]
            for j in range(1, S):
                acc = acc + a[:, 4 * i + j:4 * i + j + 1] * fts[j]
            outs.append(acc)
        return outs

    def gcn(seq_ref, a, W, bvec, alpha):
        fts = [dot(seq_ref[:, j * NIN:(j + 1) * NIN], W) for j in range(S)]
        return [_prelu(g + bvec, alpha) for g in combine(a, fts)]

    Wc = dot(dot(dot(W1_ref[...], W2_ref[...]), W3_ref[...]), Wlin_ref[...])

    def encoder1(seq_ref, a):
        g = [dot(seq_ref[:, j * NIN:(j + 1) * NIN], Wc) for j in range(S)]
        g = combine(a, combine(a, combine(a, g)))
        blin = blin_ref[...]
        a_act = a_act_ref[...]
        return [_prelu(x + blin, a_act) for x in g]

    h1 = gcn(seq1_ref, a1, W_enc2_ref[...], b_enc2_ref[...], a_enc2_ref[...])
    h2 = gcn(seq2_ref, a2, W_enc2_ref[...], b_enc2_ref[...], a_enc2_ref[...])

    third = jnp.float32(1.0 / 3.0)
    c1 = (h1[0] + h1[1] + h1[2]) * third
    mv1 = h1[3]
    ano1 = h1[2]
    c2 = (h2[0] + h2[1] + h2[2]) * third
    mv2 = h2[3]

    h3 = encoder1(seq3_ref, a1)
    h4 = encoder1(seq4_ref, a2)

    W_dec = W_dec_ref[...]
    b_dec = b_dec_ref[...]
    a_dec = a_dec_ref[...]
    d1 = combine(a1, h3)
    d2 = combine(a2, h4)
    for i in range(S):
        f1_ref[:, i * NIN:(i + 1) * NIN] = _prelu(dot(d1[i], W_dec) + b_dec, a_dec)
        f2_ref[:, i * NIN:(i + 1) * NIN] = _prelu(dot(d2[i], W_dec) + b_dec, a_dec)

    u1 = dot(mv1, Wb1_ref[...])
    u2 = dot(mv2, Wb2_ref[...])
    u3 = dot(mv1, Wb3_ref[...])
    u4 = dot(mv1, Wb4_ref[...])

    def shift_down(x):
        return jnp.concatenate([x[BB - 1:BB], x[:BB - 1]], axis=0)

    bb1 = bb1_ref[...]
    bb2 = bb2_ref[...]
    bb3 = bb3_ref[...]
    bb4 = bb4_ref[...]
    c1s = shift_down(c1)
    c2s = shift_down(c2)
    anos = shift_down(ano1)

    def rsum(x, y, b):
        return jnp.sum(x * y, axis=1, keepdims=True) + b

    sc_ref[...] = jnp.concatenate(
        [rsum(u1, c1, bb1), rsum(u1, c1s, bb1),
         rsum(u2, c2, bb2), rsum(u2, c2s, bb2),
         rsum(u3, ano1, bb3), rsum(u3, anos, bb3),
         rsum(u4, ano1, bb4), rsum(u4, anos, bb4)], axis=1)

    ex_ref[0] = jnp.concatenate(
        [u1[0:1], u2[0:1], u3[0:1], u4[0:1],
         c1[BB - 1:BB], c2[BB - 1:BB], ano1[BB - 1:BB], ano1[BB - 2:BB - 1]],
        axis=0)


def kernel(seq1, seq2, seq3, seq4, adj1, adj2,
           W_enc2, b_enc2, a_enc2, W_dec, b_dec, a_dec,
           W1, W2, W3, Wlin, blin, a_act,
           Wb1, bb1, Wb2, bb2, Wb3, bb3, Wb4, bb4):
    seqs = [x.reshape(B, S * NIN) for x in (seq1, seq2, seq3, seq4)]
    adjs = [x.reshape(B, S * S) for x in (adj1, adj2)]

    row = lambda i: (i, 0)
    whole = lambda i: (0, 0)
    seq_spec = pl.BlockSpec((BB, S * NIN), row)
    adj_spec = pl.BlockSpec((BB, S * S), row)

    def wspec(arr):
        return pl.BlockSpec(arr.shape, whole)

    weights = [W_enc2, b_enc2.reshape(1, NH), a_enc2.reshape(1, 1),
               W_dec, b_dec.reshape(1, NIN), a_dec.reshape(1, 1),
               W1, W2, W3, Wlin, blin.reshape(1, NH), a_act.reshape(1, 1),
               Wb1.reshape(NH, NH), bb1.reshape(1, 1),
               Wb2.reshape(NH, NH), bb2.reshape(1, 1),
               Wb3.reshape(NH, NH), bb3.reshape(1, 1),
               Wb4.reshape(NH, NH), bb4.reshape(1, 1)]

    f1o, f2o, sc, ex = pl.pallas_call(
        _fused_kernel,
        grid=(NB,),
        in_specs=[seq_spec] * 4 + [adj_spec] * 2 + [wspec(w) for w in weights],
        out_specs=[
            pl.BlockSpec((BB, S * NIN), row),
            pl.BlockSpec((BB, S * NIN), row),
            pl.BlockSpec((BB, 8), row),
            pl.BlockSpec((1, 8, NH), lambda i: (i, 0, 0)),
        ],
        out_shape=[
            jax.ShapeDtypeStruct((B, S * NIN), jnp.float32),
            jax.ShapeDtypeStruct((B, S * NIN), jnp.float32),
            jax.ShapeDtypeStruct((B, 8), jnp.float32),
            jax.ShapeDtypeStruct((NB, 8, NH), jnp.float32),
        ],
        compiler_params=pltpu.CompilerParams(
            dimension_semantics=("arbitrary",),
        ),
    )(*seqs, *adjs, *weights)

    u_first = ex[:, 0:4, :]
    c1_prev = jnp.roll(ex[:, 4, :], 1, axis=0)
    c2_prev = jnp.roll(ex[:, 5, :], 1, axis=0)
    ano_prev = jnp.roll(ex[:, 6, :], 1, axis=0)
    ano_prev = ano_prev.at[0].set(ex[-1, 7, :])
    fix1 = jnp.sum(u_first[:, 0] * c1_prev, axis=1) + bb1[0]
    fix2 = jnp.sum(u_first[:, 1] * c2_prev, axis=1) + bb2[0]
    fix3 = jnp.sum(u_first[:, 2] * ano_prev, axis=1) + bb3[0]
    fix4 = jnp.sum(u_first[:, 3] * ano_prev, axis=1) + bb4[0]

    sc3 = sc.reshape(NB, BB, 8)
    sc3 = (sc3.at[:, 0, 1].set(fix1).at[:, 0, 3].set(fix2)
              .at[:, 0, 5].set(fix3).at[:, 0, 7].set(fix4))
    scf = sc3.reshape(B, 8)

    def ret(ka, kb):
        return jnp.concatenate([scf[:, ka], scf[:, kb]])[:, None]

    return (ret(0, 1), ret(2, 3), ret(4, 5), ret(6, 7),
            f1o.reshape(B, S, NIN), f2o.reshape(B, S, NIN))
